# Initial kernel scaffold; baseline (speedup 1.0000x reference)
import jax
import jax.numpy as jnp
from jax.experimental import pallas as pl


def kernel(x0, edge_index0, batch0, W_gcn0, b_gcn0, W_fcg0, b_fcg0, x1, edge_index1, batch1, W_gcn1, b_gcn1, W_fcg1, b_fcg1, x2, edge_index2, batch2, W_gcn2, b_gcn2, W_fcg2, b_fcg2, x3, edge_index3, batch3, W_gcn3, b_gcn3, W_fcg3, b_fcg3, W_fc1, b_fc1, W_out, b_out):
    raise NotImplementedError("write your pallas kernel here")


# speedup vs baseline: 28.5613x; 28.5613x over previous
import functools

import jax
import jax.numpy as jnp
from jax import lax
from jax.experimental import pallas as pl
from jax.experimental.pallas import tpu as pltpu
from jax.experimental.pallas import tpu_sc as plsc

N = 10000
NPAD = 10240
E = 320000
D = 128
G = 128
NNET = 4
NC = 2
NS = 16
NW = NC * NS
CH = 64
ROWS = 158
EPT = ROWS * CH
EP = NW * EPT
RPT = NPAD // NS
NB = 10
BLK = NPAD // NB


def _sc_mesh():
    return plsc.VectorSubcoreMesh(core_axis_name="c", subcore_axis_name="s")


def _deg_body(dst_ref, deg_ref, dbuf, hist):
    c = lax.axis_index("c")
    s = lax.axis_index("s")
    wid = c * NS + s
    ones = jnp.ones((16,), jnp.float32)
    zeros = jnp.zeros((16,), jnp.float32)
    for net in range(NNET):
        pltpu.sync_copy(dst_ref.at[net, wid], dbuf)

        def _zero(i, carry):
            hist[pl.ds(i * 16, 16)] = zeros
            return carry

        lax.fori_loop(0, NPAD // 16, _zero, 0)

        def _count(r, carry):
            for k in range(CH // 16):
                idx = dbuf[r, pl.ds(k * 16, 16)]
                plsc.addupdate_scatter(hist, [idx], ones)
            return carry

        lax.fori_loop(0, ROWS, _count, 0)
        pltpu.sync_copy(hist, deg_ref.at[net, wid])


def _make_deg_kernel():
    return pl.kernel(
        _deg_body,
        out_type=jax.ShapeDtypeStruct((NNET, NW, NPAD), jnp.float32),
        mesh=_sc_mesh(),
        scratch_types=[
            pltpu.VMEM((ROWS, CH), jnp.int32),
            pltpu.VMEM((NPAD,), jnp.float32),
        ],
        compiler_params=pltpu.CompilerParams(needs_layout_passes=False),
    )


def _scatter_body(src_ref, dst_ref, g0, g1, g2, g3, acc_ref,
                  sbuf, dbuf, rowbuf, zbuf, acc, semA, semB):
    c = lax.axis_index("c")
    s = lax.axis_index("s")
    wid = c * NS + s
    gs = (g0, g1, g2, g3)
    zeros = jnp.zeros((16,), jnp.float32)
    for r in range(16):
        for k in range(8):
            zbuf[r, pl.ds(k * 16, 16)] = zeros
    buf0 = rowbuf.at[0]
    buf1 = rowbuf.at[1]
    for net in range(NNET):
        g_ref = gs[net]

        def _zero(i, carry):
            pltpu.sync_copy(zbuf, acc.at[pl.ds(s * RPT + i * 16, 16)])
            return carry

        lax.fori_loop(0, RPT // 16, _zero, 0)
        plsc.subcore_barrier()

        pltpu.sync_copy(src_ref.at[net, wid], sbuf)
        pltpu.sync_copy(dst_ref.at[net, wid], dbuf)

        def _gather(j, buf, sem):
            return pltpu.async_copy(
                g_ref.at[sbuf.at[pl.ds(j * CH, CH)]], buf, sem)

        def _wait(j, buf, sem):
            pltpu.make_async_copy(
                g_ref.at[sbuf.at[pl.ds(j * CH, CH)]], buf, sem).wait()

        _gather(0, buf0, semA)

        def _pair(i, carry):
            j0 = i * 2
            _gather(j0 + 1, buf1, semB)
            _wait(j0, buf0, semA)
            pltpu.sync_copy(buf0, acc.at[dbuf.at[j0]], add=True)

            @pl.when(j0 + 2 < ROWS)
            def _():
                _gather(j0 + 2, buf0, semA)

            _wait(j0 + 1, buf1, semB)
            pltpu.sync_copy(buf1, acc.at[dbuf.at[j0 + 1]], add=True)
            return carry

        lax.fori_loop(0, ROWS // 2, _pair, 0)

        plsc.subcore_barrier()
        pltpu.sync_copy(acc.at[pl.ds(s * RPT, RPT)],
                        acc_ref.at[net, c, pl.ds(s * RPT, RPT)])
        plsc.subcore_barrier()


def _make_scatter_kernel():
    return pl.kernel(
        _scatter_body,
        out_type=jax.ShapeDtypeStruct((NNET, NC, NPAD, D), jnp.float32),
        mesh=_sc_mesh(),
        scratch_types=[
            pltpu.VMEM((EPT,), jnp.int32),
            pltpu.VMEM((ROWS, CH), jnp.int32),
            pltpu.VMEM((2, CH, D), jnp.float32),
            pltpu.VMEM((16, D), jnp.float32),
            pltpu.VMEM_SHARED((NPAD, D), jnp.float32),
            pltpu.SemaphoreType.DMA,
            pltpu.SemaphoreType.DMA,
        ],
    )


def _conv_body(x_ref, w_ref, degp_ref, g_ref, dinv_ref):
    h = jnp.dot(x_ref[...], w_ref[...], preferred_element_type=jnp.float32)
    deg = jnp.sum(degp_ref[0], axis=0) + 1.0
    dinv = lax.rsqrt(deg)
    g_ref[...] = h * dinv[:, None]
    dinv_ref[...] = dinv


def _conv_call(net, x_p, w, deg_part):
    return pl.pallas_call(
        _conv_body,
        grid=(NB,),
        in_specs=[
            pl.BlockSpec((BLK, D), lambda b: (b, 0)),
            pl.BlockSpec((D, D), lambda b: (0, 0)),
            pl.BlockSpec((1, NW, BLK), lambda b, n=net: (n, 0, b)),
        ],
        out_specs=[
            pl.BlockSpec((BLK, D), lambda b: (b, 0)),
            pl.BlockSpec((BLK,), lambda b: (b,)),
        ],
        out_shape=[
            jax.ShapeDtypeStruct((NPAD, D), jnp.float32),
            jax.ShapeDtypeStruct((NPAD,), jnp.float32),
        ],
    )(x_p, w, deg_part)


def _pool_body(acc_ref, g_ref, dinv_ref, batch_ref, b_ref, out_ref):
    blk = pl.program_id(0)
    a = acc_ref[0, 0] + acc_ref[0, 1] + g_ref[...]
    vals = jnp.maximum(a * dinv_ref[...][:, None] + b_ref[0][None, :], 0.0)
    gids = batch_ref[...]
    gmn = batch_ref[0]
    gmx = batch_ref[BLK - 1]
    ninf = jnp.float32(-jnp.inf)

    @pl.when(blk == 0)
    def _():
        out_ref[...] = jnp.full((G, D), ninf, jnp.float32)

    gids2 = gids[:, None]
    for g in range(G):
        @pl.when((gmn <= g) & (g <= gmx))
        def _():
            mx = jnp.max(jnp.where(gids2 == g, vals, ninf), axis=0)
            out_ref[g, :] = jnp.maximum(out_ref[g, :], mx)


def _pool_call(net, acc, g, dinv, batch_p, b_gcn):
    return pl.pallas_call(
        _pool_body,
        grid=(NB,),
        in_specs=[
            pl.BlockSpec((1, NC, BLK, D), lambda b, n=net: (n, 0, b, 0)),
            pl.BlockSpec((BLK, D), lambda b: (b, 0)),
            pl.BlockSpec((BLK,), lambda b: (b,)),
            pl.BlockSpec((BLK,), lambda b: (b,)),
            pl.BlockSpec((1, D), lambda b: (0, 0)),
        ],
        out_specs=pl.BlockSpec((G, D), lambda b: (0, 0)),
        out_shape=jax.ShapeDtypeStruct((G, D), jnp.float32),
    )(acc, g, dinv, batch_p, b_gcn)


def _head_body(pooled_ref, wf_ref, bf_ref, w1_ref, b1_ref, wo_ref, bo_ref,
               out_ref):
    acc = None
    for n in range(NNET):
        t = jnp.dot(pooled_ref[n], wf_ref[n],
                    preferred_element_type=jnp.float32)
        t = jnp.maximum(t + bf_ref[n][None, :], 0.0) * 0.25
        acc = t if acc is None else acc + t
    xc = jnp.dot(acc, w1_ref[...], preferred_element_type=jnp.float32)
    xc = jnp.maximum(xc + b1_ref[...], 0.0)
    res = jnp.dot(xc, wo_ref[...], preferred_element_type=jnp.float32)
    out_ref[...] = res + bo_ref[...]


def _head_call(pooled, wf, bf, w1, b1, wo, bo):
    return pl.pallas_call(
        _head_body,
        out_shape=jax.ShapeDtypeStruct((G, D), jnp.float32),
    )(pooled, wf, bf, w1, b1, wo, bo)


def kernel(x0, edge_index0, batch0, W_gcn0, b_gcn0, W_fcg0, b_fcg0,
           x1, edge_index1, batch1, W_gcn1, b_gcn1, W_fcg1, b_fcg1,
           x2, edge_index2, batch2, W_gcn2, b_gcn2, W_fcg2, b_fcg2,
           x3, edge_index3, batch3, W_gcn3, b_gcn3, W_fcg3, b_fcg3,
           W_fc1, b_fc1, W_out, b_out):
    xs = (x0, x1, x2, x3)
    eis = (edge_index0, edge_index1, edge_index2, edge_index3)
    batches = (batch0, batch1, batch2, batch3)
    Ws = (W_gcn0, W_gcn1, W_gcn2, W_gcn3)
    bs = (b_gcn0, b_gcn1, b_gcn2, b_gcn3)
    Wf = (W_fcg0, W_fcg1, W_fcg2, W_fcg3)
    bf = (b_fcg0, b_fcg1, b_fcg2, b_fcg3)

    pad = EP - E
    ar = jnp.arange(pad, dtype=jnp.int32)
    pad_src = (ar * 37) % N
    pad_dst = N + 16 + (ar % (NPAD - N - 16))

    src_all = jnp.stack([
        jnp.concatenate([ei[0], pad_src]).reshape(NW, EPT) for ei in eis])
    dst_all = jnp.stack([
        jnp.concatenate([ei[1], pad_dst]).reshape(NW, ROWS, CH)
        for ei in eis])

    deg_part = _make_deg_kernel()(dst_all)

    x_ps = [jnp.pad(x, ((0, NPAD - N), (0, 0))) for x in xs]
    gs, dinvs = [], []
    for i in range(NNET):
        g_i, dinv_i = _conv_call(i, x_ps[i], Ws[i], deg_part)
        gs.append(g_i)
        dinvs.append(dinv_i)

    acc = _make_scatter_kernel()(src_all, dst_all, *gs)

    pad_b = jnp.full((NPAD - N,), G, jnp.int32)
    pooled = []
    for i in range(NNET):
        batch_p = jnp.concatenate([batches[i], pad_b])
        pooled.append(_pool_call(i, acc, gs[i], dinvs[i], batch_p,
                                 bs[i].reshape(1, D)))

    pooled_s = jnp.stack(pooled)
    wf_s = jnp.stack(Wf)
    bf_s = jnp.stack(bf)
    wo_p = jnp.pad(W_out, ((0, 0), (0, D - 1)))
    bo_p = jnp.pad(b_out, (0, D - 1)).reshape(1, D)
    res = _head_call(pooled_s, wf_s, bf_s, W_fc1, b_fc1.reshape(1, 64),
                     wo_p, bo_p)
    return res[:, :1]


# speedup vs baseline: 33.2663x; 1.1647x over previous
import functools

import jax
import jax.numpy as jnp
from jax import lax
from jax.experimental import pallas as pl
from jax.experimental.pallas import tpu as pltpu
from jax.experimental.pallas import tpu_sc as plsc

N = 10000
NPAD = 10240
E = 320000
D = 128
G = 128
NNET = 4
NC = 2
NS = 16
NW = NC * NS
CH = 64
ROWS = 158
EPT = ROWS * CH
EP = NW * EPT
RPT = NPAD // NS
NB = 10
BLK = NPAD // NB


def _sc_mesh():
    return plsc.VectorSubcoreMesh(core_axis_name="c", subcore_axis_name="s")


def _deg_body(dst_ref, deg_ref, dbuf, hist):
    c = lax.axis_index("c")
    s = lax.axis_index("s")
    wid = c * NS + s
    ones = jnp.ones((16,), jnp.float32)
    zeros = jnp.zeros((16,), jnp.float32)
    for net in range(NNET):
        pltpu.sync_copy(dst_ref.at[net, wid], dbuf)

        def _zero(i, carry):
            hist[pl.ds(i * 16, 16)] = zeros
            return carry

        lax.fori_loop(0, NPAD // 16, _zero, 0)

        def _count(r, carry):
            for k in range(CH // 16):
                idx = dbuf[r, pl.ds(k * 16, 16)]
                plsc.addupdate_scatter(hist, [idx], ones)
            return carry

        lax.fori_loop(0, ROWS, _count, 0)
        pltpu.sync_copy(hist, deg_ref.at[net, wid])


def _make_deg_kernel():
    return pl.kernel(
        _deg_body,
        out_type=jax.ShapeDtypeStruct((NNET, NW, NPAD), jnp.float32),
        mesh=_sc_mesh(),
        scratch_types=[
            pltpu.VMEM((ROWS, CH), jnp.int32),
            pltpu.VMEM((NPAD,), jnp.float32),
        ],
        compiler_params=pltpu.CompilerParams(needs_layout_passes=False),
    )


def _scatter_body(src_ref, dst_ref, g_ref, acc_ref,
                  sbuf, dbuf, rowbuf, zbuf, acc, semA, semB):
    c = lax.axis_index("c")
    s = lax.axis_index("s")
    wid = c * NS + s
    zeros = jnp.zeros((16,), jnp.float32)
    for r in range(16):
        for k in range(8):
            zbuf[r, pl.ds(k * 16, 16)] = zeros
    buf0 = rowbuf.at[0]
    buf1 = rowbuf.at[1]

    def _zero(i, carry):
        pltpu.sync_copy(zbuf, acc.at[pl.ds(s * RPT + i * 16, 16)])
        return carry

    lax.fori_loop(0, RPT // 16, _zero, 0)
    plsc.subcore_barrier()

    pltpu.sync_copy(src_ref.at[wid], sbuf)
    pltpu.sync_copy(dst_ref.at[wid], dbuf)

    def _gather(j, buf, sem):
        return pltpu.async_copy(
            g_ref.at[sbuf.at[pl.ds(j * CH, CH)]], buf, sem)

    def _wait(j, buf, sem):
        pltpu.make_async_copy(
            g_ref.at[sbuf.at[pl.ds(j * CH, CH)]], buf, sem).wait()

    _gather(0, buf0, semA)

    def _pair(i, carry):
        j0 = i * 2
        _gather(j0 + 1, buf1, semB)
        _wait(j0, buf0, semA)
        pltpu.sync_copy(buf0, acc.at[dbuf.at[j0]], add=True)

        @pl.when(j0 + 2 < ROWS)
        def _():
            _gather(j0 + 2, buf0, semA)

        _wait(j0 + 1, buf1, semB)
        pltpu.sync_copy(buf1, acc.at[dbuf.at[j0 + 1]], add=True)
        return carry

    lax.fori_loop(0, ROWS // 2, _pair, 0)

    plsc.subcore_barrier()
    pltpu.sync_copy(acc.at[pl.ds(s * RPT, RPT)],
                    acc_ref.at[c, pl.ds(s * RPT, RPT)])


def _make_scatter_kernel():
    return pl.kernel(
        _scatter_body,
        out_type=jax.ShapeDtypeStruct((NC, NPAD, D), jnp.float32),
        mesh=_sc_mesh(),
        scratch_types=[
            pltpu.VMEM((EPT,), jnp.int32),
            pltpu.VMEM((ROWS, CH), jnp.int32),
            pltpu.VMEM((2, CH, D), jnp.float32),
            pltpu.VMEM((16, D), jnp.float32),
            pltpu.VMEM_SHARED((NPAD, D), jnp.float32),
            pltpu.SemaphoreType.DMA,
            pltpu.SemaphoreType.DMA,
        ],
    )


def _conv_body(x_ref, w_ref, degp_ref, g_ref, dinv_ref):
    h = jnp.dot(x_ref[...], w_ref[...], preferred_element_type=jnp.float32)
    deg = jnp.sum(degp_ref[0], axis=0) + 1.0
    dinv = lax.rsqrt(deg)
    g_ref[...] = h * dinv[:, None]
    dinv_ref[...] = dinv


def _conv_call(net, x_p, w, deg_part):
    return pl.pallas_call(
        _conv_body,
        grid=(NB,),
        in_specs=[
            pl.BlockSpec((BLK, D), lambda b: (b, 0)),
            pl.BlockSpec((D, D), lambda b: (0, 0)),
            pl.BlockSpec((1, NW, BLK), lambda b, n=net: (n, 0, b)),
        ],
        out_specs=[
            pl.BlockSpec((BLK, D), lambda b: (b, 0)),
            pl.BlockSpec((BLK,), lambda b: (b,)),
        ],
        out_shape=[
            jax.ShapeDtypeStruct((NPAD, D), jnp.float32),
            jax.ShapeDtypeStruct((NPAD,), jnp.float32),
        ],
    )(x_p, w, deg_part)


def _pool_body(acc_ref, g_ref, dinv_ref, batch_ref, b_ref, out_ref):
    blk = pl.program_id(0)
    a = acc_ref[0] + acc_ref[1] + g_ref[...]
    vals = jnp.maximum(a * dinv_ref[...][:, None] + b_ref[0][None, :], 0.0)
    gids = batch_ref[...]
    gmn = batch_ref[0]
    gmx = batch_ref[BLK - 1]
    ninf = jnp.float32(-jnp.inf)

    @pl.when(blk == 0)
    def _():
        out_ref[...] = jnp.full((G, D), ninf, jnp.float32)

    gids2 = gids[:, None]
    for g in range(G):
        @pl.when((gmn <= g) & (g <= gmx))
        def _():
            mx = jnp.max(jnp.where(gids2 == g, vals, ninf), axis=0)
            out_ref[g, :] = jnp.maximum(out_ref[g, :], mx)


def _pool_call(acc, g, dinv, batch_p, b_gcn):
    return pl.pallas_call(
        _pool_body,
        grid=(NB,),
        in_specs=[
            pl.BlockSpec((NC, BLK, D), lambda b: (0, b, 0)),
            pl.BlockSpec((BLK, D), lambda b: (b, 0)),
            pl.BlockSpec((BLK,), lambda b: (b,)),
            pl.BlockSpec((BLK,), lambda b: (b,)),
            pl.BlockSpec((1, D), lambda b: (0, 0)),
        ],
        out_specs=pl.BlockSpec((G, D), lambda b: (0, 0)),
        out_shape=jax.ShapeDtypeStruct((G, D), jnp.float32),
    )(acc, g, dinv, batch_p, b_gcn)


def _head_body(pooled_ref, wf_ref, bf_ref, w1_ref, b1_ref, wo_ref, bo_ref,
               out_ref):
    acc = None
    for n in range(NNET):
        t = jnp.dot(pooled_ref[n], wf_ref[n],
                    preferred_element_type=jnp.float32)
        t = jnp.maximum(t + bf_ref[n][None, :], 0.0) * 0.25
        acc = t if acc is None else acc + t
    xc = jnp.dot(acc, w1_ref[...], preferred_element_type=jnp.float32)
    xc = jnp.maximum(xc + b1_ref[...], 0.0)
    res = jnp.dot(xc, wo_ref[...], preferred_element_type=jnp.float32)
    out_ref[...] = res + bo_ref[...]


def _head_call(pooled, wf, bf, w1, b1, wo, bo):
    return pl.pallas_call(
        _head_body,
        out_shape=jax.ShapeDtypeStruct((G, D), jnp.float32),
    )(pooled, wf, bf, w1, b1, wo, bo)


def kernel(x0, edge_index0, batch0, W_gcn0, b_gcn0, W_fcg0, b_fcg0,
           x1, edge_index1, batch1, W_gcn1, b_gcn1, W_fcg1, b_fcg1,
           x2, edge_index2, batch2, W_gcn2, b_gcn2, W_fcg2, b_fcg2,
           x3, edge_index3, batch3, W_gcn3, b_gcn3, W_fcg3, b_fcg3,
           W_fc1, b_fc1, W_out, b_out):
    xs = (x0, x1, x2, x3)
    eis = (edge_index0, edge_index1, edge_index2, edge_index3)
    batches = (batch0, batch1, batch2, batch3)
    Ws = (W_gcn0, W_gcn1, W_gcn2, W_gcn3)
    bs = (b_gcn0, b_gcn1, b_gcn2, b_gcn3)
    Wf = (W_fcg0, W_fcg1, W_fcg2, W_fcg3)
    bf = (b_fcg0, b_fcg1, b_fcg2, b_fcg3)

    pad = EP - E
    ar = jnp.arange(pad, dtype=jnp.int32)
    pad_src = (ar * 37) % N
    pad_dst = N + 16 + (ar % (NPAD - N - 16))

    srcs = [jnp.concatenate([ei[0], pad_src]).reshape(NW, EPT) for ei in eis]
    dsts = [jnp.concatenate([ei[1], pad_dst]).reshape(NW, ROWS, CH)
            for ei in eis]
    dst_all = jnp.stack(dsts)

    deg_part = _make_deg_kernel()(dst_all)

    x_ps = [jnp.pad(x, ((0, NPAD - N), (0, 0))) for x in xs]
    gs, dinvs = [], []
    for i in range(NNET):
        g_i, dinv_i = _conv_call(i, x_ps[i], Ws[i], deg_part)
        gs.append(g_i)
        dinvs.append(dinv_i)

    scatter = _make_scatter_kernel()
    pad_b = jnp.full((NPAD - N,), G, jnp.int32)
    pooled = []
    for i in range(NNET):
        acc_i = scatter(srcs[i], dsts[i], gs[i])
        batch_p = jnp.concatenate([batches[i], pad_b])
        pooled.append(_pool_call(acc_i, gs[i], dinvs[i], batch_p,
                                 bs[i].reshape(1, D)))

    pooled_s = jnp.stack(pooled)
    wf_s = jnp.stack(Wf)
    bf_s = jnp.stack(bf)
    wo_p = jnp.pad(W_out, ((0, 0), (0, D - 1)))
    bo_p = jnp.pad(b_out, (0, D - 1)).reshape(1, D)
    res = _head_call(pooled_s, wf_s, bf_s, W_fc1, b_fc1.reshape(1, 64),
                     wo_p, bo_p)
    return res[:, :1]


# speedup vs baseline: 36.7074x; 1.1034x over previous
import functools

import jax
import jax.numpy as jnp
from jax import lax
from jax.experimental import pallas as pl
from jax.experimental.pallas import tpu as pltpu
from jax.experimental.pallas import tpu_sc as plsc

N = 10000
NPAD = 10240
E = 320000
D = 128
G = 128
NNET = 4
NC = 2
NS = 16
NW = NC * NS
CH = 64
ROWS = 160
HR = ROWS // 2
EPT = ROWS * CH
EP = NW * EPT
RPT = NPAD // NS
NB = 10
BLK = NPAD // NB


def _sc_mesh():
    return plsc.VectorSubcoreMesh(core_axis_name="c", subcore_axis_name="s")


def _deg_body(dst_ref, deg_ref, dbuf, hist):
    c = lax.axis_index("c")
    s = lax.axis_index("s")
    wid = c * NS + s
    ones = jnp.ones((16,), jnp.float32)
    zeros = jnp.zeros((16,), jnp.float32)
    for net in range(NNET):
        pltpu.sync_copy(dst_ref.at[net, wid], dbuf)

        def _zero(i, carry):
            hist[pl.ds(i * 16, 16)] = zeros
            return carry

        lax.fori_loop(0, NPAD // 16, _zero, 0)

        def _count(r, carry):
            for k in range(CH // 16):
                idx = dbuf[r, pl.ds(k * 16, 16)]
                plsc.addupdate_scatter(hist, [idx], ones)
            return carry

        lax.fori_loop(0, ROWS, _count, 0)
        pltpu.sync_copy(hist, deg_ref.at[net, wid])


def _make_deg_kernel():
    return pl.kernel(
        _deg_body,
        out_type=jax.ShapeDtypeStruct((NNET, NW, NPAD), jnp.float32),
        mesh=_sc_mesh(),
        scratch_types=[
            pltpu.VMEM((ROWS, CH), jnp.int32),
            pltpu.VMEM((NPAD,), jnp.float32),
        ],
        compiler_params=pltpu.CompilerParams(needs_layout_passes=False),
    )


def _scatter_body(src_ref, dst_ref, g_ref, acc_ref,
                  sbuf, dbuf, rowbuf, zbuf, acc, sGa, sSa):
    c = lax.axis_index("c")
    s = lax.axis_index("s")
    wid = c * NS + s
    zeros = jnp.zeros((16,), jnp.float32)
    for r in range(16):
        for k in range(8):
            zbuf[r, pl.ds(k * 16, 16)] = zeros
    bufs = (rowbuf.at[0], rowbuf.at[1], rowbuf.at[2])
    sG = (sGa.at[0], sGa.at[1], sGa.at[2])
    sS = (sSa.at[0], sSa.at[1], sSa.at[2])

    def _zero(i, carry):
        pltpu.sync_copy(zbuf, acc.at[pl.ds(s * RPT + i * 16, 16)])
        return carry

    lax.fori_loop(0, RPT // 16, _zero, 0)
    plsc.subcore_barrier()

    def _gather(j, k):
        pltpu.async_copy(g_ref.at[sbuf.at[pl.ds(j * CH, CH)]], bufs[k],
                         sG[k])

    def _gwait(j, k):
        pltpu.make_async_copy(g_ref.at[sbuf.at[pl.ds(j * CH, CH)]],
                              bufs[k], sG[k]).wait()

    def _scat(j, k):
        pltpu.async_copy(bufs[k], acc.at[dbuf.at[j]], sS[k], add=True)

    def _swait(j, k):
        pltpu.make_async_copy(bufs[k], acc.at[dbuf.at[j]], sS[k]).wait()

    for h in range(2):
        pltpu.sync_copy(src_ref.at[wid, pl.ds(h * HR * CH, HR * CH)], sbuf)
        pltpu.sync_copy(dst_ref.at[wid, pl.ds(h * HR, HR)], dbuf)

        _gather(0, 0)
        _gather(1, 1)

        def _round(r, carry):
            j0 = r * 3
            for k in range(3):
                j = j0 + k
                _gwait(j, k)
                _scat(j, k)
                km1 = (k - 1) % 3
                if k == 0:
                    @pl.when(j > 0)
                    def _():
                        _swait(j - 1, km1)
                else:
                    _swait(j - 1, km1)
                _gather(j + 2, km1)
            return carry

        nfull = (HR - 2) // 3
        lax.fori_loop(0, nfull, _round, 0)
        for j in range(nfull * 3, HR):
            k = j % 3
            _gwait(j, k)
            _scat(j, k)
            _swait(j - 1, (k - 1) % 3)
        _swait(HR - 1, (HR - 1) % 3)

    plsc.subcore_barrier()
    pltpu.sync_copy(acc.at[pl.ds(s * RPT, RPT)],
                    acc_ref.at[c, pl.ds(s * RPT, RPT)])


def _make_scatter_kernel():
    return pl.kernel(
        _scatter_body,
        out_type=jax.ShapeDtypeStruct((NC, NPAD, D), jnp.float32),
        mesh=_sc_mesh(),
        scratch_types=[
            pltpu.VMEM((HR * CH,), jnp.int32),
            pltpu.VMEM((HR, CH), jnp.int32),
            pltpu.VMEM((3, CH, D), jnp.float32),
            pltpu.VMEM((16, D), jnp.float32),
            pltpu.VMEM_SHARED((NPAD, D), jnp.float32),
            pltpu.SemaphoreType.DMA((3,)),
            pltpu.SemaphoreType.DMA((3,)),
        ],
    )


def _conv_body(x_ref, w_ref, degp_ref, g_ref, dinv_ref):
    h = jnp.dot(x_ref[...], w_ref[...], preferred_element_type=jnp.float32)
    deg = jnp.sum(degp_ref[0], axis=0) + 1.0
    dinv = lax.rsqrt(deg)
    g_ref[...] = h * dinv[:, None]
    dinv_ref[...] = dinv


def _conv_call(net, x_p, w, deg_part):
    return pl.pallas_call(
        _conv_body,
        grid=(NB,),
        in_specs=[
            pl.BlockSpec((BLK, D), lambda b: (b, 0)),
            pl.BlockSpec((D, D), lambda b: (0, 0)),
            pl.BlockSpec((1, NW, BLK), lambda b, n=net: (n, 0, b)),
        ],
        out_specs=[
            pl.BlockSpec((BLK, D), lambda b: (b, 0)),
            pl.BlockSpec((BLK,), lambda b: (b,)),
        ],
        out_shape=[
            jax.ShapeDtypeStruct((NPAD, D), jnp.float32),
            jax.ShapeDtypeStruct((NPAD,), jnp.float32),
        ],
    )(x_p, w, deg_part)


def _pool_body(acc_ref, g_ref, dinv_ref, batch_ref, b_ref, out_ref):
    blk = pl.program_id(0)
    a = acc_ref[0] + acc_ref[1] + g_ref[...]
    vals = jnp.maximum(a * dinv_ref[...][:, None] + b_ref[0][None, :], 0.0)
    gids = batch_ref[...]
    gmn = batch_ref[0]
    gmx = batch_ref[BLK - 1]
    ninf = jnp.float32(-jnp.inf)

    @pl.when(blk == 0)
    def _():
        out_ref[...] = jnp.full((G, D), ninf, jnp.float32)

    gids2 = gids[:, None]
    for g in range(G):
        @pl.when((gmn <= g) & (g <= gmx))
        def _():
            mx = jnp.max(jnp.where(gids2 == g, vals, ninf), axis=0)
            out_ref[g, :] = jnp.maximum(out_ref[g, :], mx)


def _pool_call(acc, g, dinv, batch_p, b_gcn):
    return pl.pallas_call(
        _pool_body,
        grid=(NB,),
        in_specs=[
            pl.BlockSpec((NC, BLK, D), lambda b: (0, b, 0)),
            pl.BlockSpec((BLK, D), lambda b: (b, 0)),
            pl.BlockSpec((BLK,), lambda b: (b,)),
            pl.BlockSpec((BLK,), lambda b: (b,)),
            pl.BlockSpec((1, D), lambda b: (0, 0)),
        ],
        out_specs=pl.BlockSpec((G, D), lambda b: (0, 0)),
        out_shape=jax.ShapeDtypeStruct((G, D), jnp.float32),
    )(acc, g, dinv, batch_p, b_gcn)


def _head_body(pooled_ref, wf_ref, bf_ref, w1_ref, b1_ref, wo_ref, bo_ref,
               out_ref):
    acc = None
    for n in range(NNET):
        t = jnp.dot(pooled_ref[n], wf_ref[n],
                    preferred_element_type=jnp.float32)
        t = jnp.maximum(t + bf_ref[n][None, :], 0.0) * 0.25
        acc = t if acc is None else acc + t
    xc = jnp.dot(acc, w1_ref[...], preferred_element_type=jnp.float32)
    xc = jnp.maximum(xc + b1_ref[...], 0.0)
    res = jnp.dot(xc, wo_ref[...], preferred_element_type=jnp.float32)
    out_ref[...] = res + bo_ref[...]


def _head_call(pooled, wf, bf, w1, b1, wo, bo):
    return pl.pallas_call(
        _head_body,
        out_shape=jax.ShapeDtypeStruct((G, D), jnp.float32),
    )(pooled, wf, bf, w1, b1, wo, bo)


def kernel(x0, edge_index0, batch0, W_gcn0, b_gcn0, W_fcg0, b_fcg0,
           x1, edge_index1, batch1, W_gcn1, b_gcn1, W_fcg1, b_fcg1,
           x2, edge_index2, batch2, W_gcn2, b_gcn2, W_fcg2, b_fcg2,
           x3, edge_index3, batch3, W_gcn3, b_gcn3, W_fcg3, b_fcg3,
           W_fc1, b_fc1, W_out, b_out):
    xs = (x0, x1, x2, x3)
    eis = (edge_index0, edge_index1, edge_index2, edge_index3)
    batches = (batch0, batch1, batch2, batch3)
    Ws = (W_gcn0, W_gcn1, W_gcn2, W_gcn3)
    bs = (b_gcn0, b_gcn1, b_gcn2, b_gcn3)
    Wf = (W_fcg0, W_fcg1, W_fcg2, W_fcg3)
    bf = (b_fcg0, b_fcg1, b_fcg2, b_fcg3)

    pad = EP - E
    ar = jnp.arange(pad, dtype=jnp.int32)
    pad_src = (ar * 37) % N
    pad_dst = N + 16 + (ar % (NPAD - N - 16))

    srcs = [jnp.concatenate([ei[0], pad_src]).reshape(NW, EPT) for ei in eis]
    dsts = [jnp.concatenate([ei[1], pad_dst]).reshape(NW, ROWS, CH)
            for ei in eis]
    dst_all = jnp.stack(dsts)

    deg_part = _make_deg_kernel()(dst_all)

    x_ps = [jnp.pad(x, ((0, NPAD - N), (0, 0))) for x in xs]
    gs, dinvs = [], []
    for i in range(NNET):
        g_i, dinv_i = _conv_call(i, x_ps[i], Ws[i], deg_part)
        gs.append(g_i)
        dinvs.append(dinv_i)

    scatter = _make_scatter_kernel()
    pad_b = jnp.full((NPAD - N,), G, jnp.int32)
    pooled = []
    for i in range(NNET):
        acc_i = scatter(srcs[i], dsts[i], gs[i])
        batch_p = jnp.concatenate([batches[i], pad_b])
        pooled.append(_pool_call(acc_i, gs[i], dinvs[i], batch_p,
                                 bs[i].reshape(1, D)))

    pooled_s = jnp.stack(pooled)
    wf_s = jnp.stack(Wf)
    bf_s = jnp.stack(bf)
    wo_p = jnp.pad(W_out, ((0, 0), (0, D - 1)))
    bo_p = jnp.pad(b_out, (0, D - 1)).reshape(1, D)
    res = _head_call(pooled_s, wf_s, bf_s, W_fc1, b_fc1.reshape(1, 64),
                     wo_p, bo_p)
    return res[:, :1]
